# baseline (speedup 1.0000x reference)
import functools

import jax
import jax.numpy as jnp
from jax import lax
from jax.experimental import pallas as pl
from jax.experimental.pallas import tpu as pltpu
from jax.experimental.pallas import tpu_sc as plsc

K = 64
D = 4096
NW_USED = 8
ROWS_PER_W = K // NW_USED

_mesh = plsc.VectorSubcoreMesh(core_axis_name="c", subcore_axis_name="s")


@functools.partial(
    pl.kernel,
    mesh=_mesh,
    out_type=jax.ShapeDtypeStruct((K, D), jnp.float32),
    scratch_types=[
        pltpu.VMEM((ROWS_PER_W,), jnp.int32),
        pltpu.VMEM((ROWS_PER_W, D), jnp.float32),
        pltpu.SemaphoreType.DMA,
    ],
)
def _gather(table_hbm, ids_hbm, out_hbm, idx_v, rows_v, sem):
    info = plsc.get_sparse_core_info()
    wid = lax.axis_index("s") * info.num_cores + lax.axis_index("c")

    @pl.when(wid < NW_USED)
    def _():
        base = wid * ROWS_PER_W
        pltpu.sync_copy(ids_hbm.at[pl.ds(base, ROWS_PER_W)], idx_v)
        pltpu.async_copy(table_hbm.at[idx_v], rows_v, sem).wait()
        pltpu.sync_copy(rows_v, out_hbm.at[pl.ds(base, ROWS_PER_W)])


def kernel(table, position_ids):
    return _gather(table, position_ids.astype(jnp.int32))
